# baseline (speedup 1.0000x reference)
import functools

import jax
import jax.numpy as jnp
from jax import lax
from jax.experimental import pallas as pl
from jax.experimental.pallas import tpu as pltpu
from jax.experimental.pallas import tpu_sc as plsc

_NUM_CLASSES = 1000
_BATCH = 16384
_NC = 2
_NS = 16
_NW = _NC * _NS
_L = 16
_ROWS_PER_W = _BATCH // _NW
_FLAT_PER_W = _ROWS_PER_W * _NUM_CLASSES
_ZBUF_WORDS = 64000
_NFILL = _FLAT_PER_W // _ZBUF_WORDS
_NSCAT = _ROWS_PER_W // 128


@functools.partial(
    pl.kernel,
    out_type=jax.ShapeDtypeStruct((_BATCH * _NUM_CLASSES,), jnp.float32),
    mesh=plsc.VectorSubcoreMesh(core_axis_name="c", subcore_axis_name="s"),
    scratch_types=[
        pltpu.VMEM((_ZBUF_WORDS,), jnp.float32),
        pltpu.VMEM((_ROWS_PER_W,), jnp.int32),
        pltpu.VMEM((_NSCAT, 128), jnp.int32),
        pltpu.VMEM((128,), jnp.float32),
        pltpu.SemaphoreType.DMA,
        pltpu.SemaphoreType.DMA,
    ],
)
def _onehot_sc(x_hbm, out_hbm, zbuf, idx_v, pos_v, ones_v, fill_sem, scat_sem):
    wid = lax.axis_index("s") * _NC + lax.axis_index("c")
    base_row = wid * _ROWS_PER_W
    flat_base = wid * _FLAT_PER_W

    pltpu.sync_copy(x_hbm.at[pl.ds(base_row, _ROWS_PER_W)], idx_v)

    zvec = jnp.zeros((_L,), jnp.float32)

    def _zero_body(j, c):
        base = j * (_L * 16)
        for k in range(16):
            zbuf[pl.ds(base + k * _L, _L)] = zvec
        return c

    lax.fori_loop(0, _ZBUF_WORDS // (_L * 16), _zero_body, 0)

    for i in range(_NFILL):
        pltpu.make_async_copy(
            zbuf,
            out_hbm.at[pl.ds(flat_base + i * _ZBUF_WORDS, _ZBUF_WORDS)],
            fill_sem,
        ).start()

    row_iota = lax.iota(jnp.int32, _L)
    onevec = jnp.ones((_L,), jnp.float32)
    for j in range(_ROWS_PER_W // _L):
        xv = idx_v[pl.ds(j * _L, _L)]
        pos = (base_row + j * _L + row_iota) * _NUM_CLASSES + xv
        pos_v[j // 8, pl.ds((j % 8) * _L, _L)] = pos
    for k in range(128 // _L):
        ones_v[pl.ds(k * _L, _L)] = onevec

    for i in range(_NFILL):
        pltpu.make_async_copy(
            zbuf,
            out_hbm.at[pl.ds(flat_base + i * _ZBUF_WORDS, _ZBUF_WORDS)],
            fill_sem,
        ).wait()

    for j in range(_NSCAT):
        pltpu.make_async_copy(ones_v, out_hbm.at[pos_v.at[j]], scat_sem).start()
    for j in range(_NSCAT):
        pltpu.make_async_copy(ones_v, out_hbm.at[pos_v.at[j]], scat_sem).wait()


def kernel(x):
    xi = x.astype(jnp.int32)
    flat = _onehot_sc(xi)
    return flat.reshape(_BATCH, _NUM_CLASSES)


# speedup vs baseline: 1.7664x; 1.7664x over previous
import functools

import jax
import jax.numpy as jnp
from jax import lax
from jax.experimental import pallas as pl
from jax.experimental.pallas import tpu as pltpu
from jax.experimental.pallas import tpu_sc as plsc

_NUM_CLASSES = 1000
_BATCH = 16384
_NC = 2
_NS = 16
_NW = _NC * _NS
_L = 16
_ROWS_PER_W = _BATCH // _NW
_CHUNK = 32
_NCHUNK = _ROWS_PER_W // _CHUNK
_NBUF = 2


@functools.partial(
    pl.kernel,
    out_type=jax.ShapeDtypeStruct((_BATCH, _NUM_CLASSES), jnp.float32),
    mesh=plsc.VectorSubcoreMesh(core_axis_name="c", subcore_axis_name="s"),
    scratch_types=[
        pltpu.VMEM((_NBUF * _CHUNK, _NUM_CLASSES), jnp.float32),
        pltpu.VMEM((_ROWS_PER_W,), jnp.int32),
        [pltpu.SemaphoreType.DMA] * _NBUF,
    ],
    compiler_params=pltpu.CompilerParams(
        use_tc_tiling_on_sc=True, needs_layout_passes=False
    ),
)
def _onehot_sc(x_hbm, out_hbm, bufs, idx_v, sems):
    wid = lax.axis_index("s") * _NC + lax.axis_index("c")
    base_row = wid * _ROWS_PER_W

    pltpu.sync_copy(x_hbm.at[pl.ds(base_row, _ROWS_PER_W)], idx_v)

    zvec = jnp.zeros((_L,), jnp.float32)

    def _zero_body(r, c):
        for k in range(_NUM_CLASSES // _L):
            bufs[r, pl.ds(k * _L, _L)] = zvec
        bufs[r, pl.ds(_NUM_CLASSES - _L, _L)] = zvec
        return c

    lax.fori_loop(0, _NBUF * _CHUNK, _zero_body, 0)

    row_iota = lax.iota(jnp.int32, _L)
    onevec = jnp.ones((_L,), jnp.float32)

    def _positions(b, c):
        out = []
        for j in range(_CHUNK // _L):
            rows = b * _CHUNK + j * _L + row_iota
            cols = idx_v[pl.ds(c * _CHUNK + j * _L, _L)]
            out.append((rows, cols))
        return out

    def _dma(b, c):
        return pltpu.make_async_copy(
            bufs.at[pl.ds(b * _CHUNK, _CHUNK)],
            out_hbm.at[pl.ds(base_row + c * _CHUNK, _CHUNK)],
            sems[b],
        )

    for c in range(_NCHUNK):
        b = c % _NBUF
        if c >= _NBUF:
            _dma(b, c - _NBUF).wait()
            for rows, cols in _positions(b, c - _NBUF):
                plsc.store_scatter(bufs, [rows, cols], zvec)
        for rows, cols in _positions(b, c):
            plsc.store_scatter(bufs, [rows, cols], onevec)
        _dma(b, c).start()

    for c in range(_NCHUNK - _NBUF, _NCHUNK):
        _dma(c % _NBUF, c).wait()


def kernel(x):
    xi = x.astype(jnp.int32)
    return _onehot_sc(xi)


# speedup vs baseline: 3.9829x; 2.2548x over previous
import functools

import jax
import jax.numpy as jnp
from jax import lax
from jax.experimental import pallas as pl
from jax.experimental.pallas import tpu as pltpu
from jax.experimental.pallas import tpu_sc as plsc

_NUM_CLASSES = 1000
_BATCH = 16384
_NC = 2
_NS = 16
_NW = _NC * _NS
_L = 16
_COLS_PER_W = _BATCH // _NW
_CHUNK = 128
_NCHUNK = _COLS_PER_W // _CHUNK


@functools.partial(
    pl.kernel,
    out_type=jax.ShapeDtypeStruct((_NUM_CLASSES, _BATCH), jnp.float32),
    mesh=plsc.VectorSubcoreMesh(core_axis_name="c", subcore_axis_name="s"),
    scratch_types=[
        pltpu.VMEM((_NUM_CLASSES, _CHUNK), jnp.float32),
        pltpu.VMEM((_COLS_PER_W,), jnp.int32),
        pltpu.SemaphoreType.DMA,
    ],
    compiler_params=pltpu.CompilerParams(
        use_tc_tiling_on_sc=True, needs_layout_passes=False
    ),
)
def _onehot_sc(x_hbm, out_hbm, buf, idx_v, sem):
    wid = lax.axis_index("s") * _NC + lax.axis_index("c")
    base_col = wid * _COLS_PER_W

    pltpu.sync_copy(x_hbm.at[pl.ds(base_col, _COLS_PER_W)], idx_v)

    zvec = jnp.zeros((_L,), jnp.float32)

    def _zero_body(r, c):
        for k in range(4):
            for j in range(_CHUNK // _L):
                buf[r * 4 + k, pl.ds(j * _L, _L)] = zvec
        return c

    lax.fori_loop(0, _NUM_CLASSES // 4, _zero_body, 0)

    lane_iota = lax.iota(jnp.int32, _L)
    onevec = jnp.ones((_L,), jnp.float32)

    def _positions(c):
        out = []
        for j in range(_CHUNK // _L):
            rows = idx_v[pl.ds(c * _CHUNK + j * _L, _L)]
            cols = j * _L + lane_iota
            out.append((rows, cols))
        return out

    for c in range(_NCHUNK):
        for rows, cols in _positions(c):
            plsc.store_scatter(buf, [rows, cols], onevec)
        copy = pltpu.make_async_copy(
            buf,
            out_hbm.at[:, pl.ds(base_col + c * _CHUNK, _CHUNK)],
            sem,
        )
        copy.start()
        copy.wait()
        for rows, cols in _positions(c):
            plsc.store_scatter(buf, [rows, cols], zvec)


def kernel(x):
    xi = x.astype(jnp.int32)
    return _onehot_sc(xi).T
